# baseline (speedup 1.0000x reference)
import functools

import jax
import jax.numpy as jnp
from jax import lax
from jax.experimental import pallas as pl
from jax.experimental.pallas import tpu as pltpu
from jax.experimental.pallas import tpu_sc as plsc

R, C = 128, 32768
L = 16
NC, NS = 2, 16
NW = NC * NS
ROWS_PER_W = R // NW
U = 8


def _body(in_hbm, out_hbm, row_f, out_v, red16):
    wid = lax.axis_index("s") * NC + lax.axis_index("c")

    neg_inf = jnp.full((L,), -jnp.inf, jnp.float32)
    one = jnp.full((L,), 1, jnp.int32)
    zero = jnp.full((L,), 0, jnp.int32)

    n_chunks = C // (L * U)

    for i in range(ROWS_PER_W):
        row = wid * ROWS_PER_W + i
        pltpu.sync_copy(in_hbm.at[row], row_f)

        def max_body(k, accs):
            base = k * (L * U)
            return tuple(
                jnp.maximum(a, row_f[pl.ds(base + j * L, L)])
                for j, a in enumerate(accs)
            )

        accs = lax.fori_loop(0, n_chunks, max_body, (neg_inf,) * U)
        acc = functools.reduce(jnp.maximum, accs)
        srt, _ = plsc.sort_key_val(acc, acc)
        red16[...] = srt
        mxv = plsc.load_gather(red16, [jnp.full((L,), L - 1, jnp.int32)])

        def cmp_body(k, carry):
            base = k * (L * U)
            for j in range(U):
                v = row_f[pl.ds(base + j * L, L)]
                out_v[pl.ds(base + j * L, L)] = jnp.where(v == mxv, one, zero)
            return carry

        lax.fori_loop(0, n_chunks, cmp_body, 0)
        pltpu.sync_copy(out_v, out_hbm.at[row])


def kernel(input):
    mesh = plsc.VectorSubcoreMesh(core_axis_name="c", subcore_axis_name="s")
    k = pl.kernel(
        _body,
        out_type=jax.ShapeDtypeStruct((R, C), jnp.int32),
        mesh=mesh,
        scratch_types=[
            pltpu.VMEM((C,), jnp.float32),
            pltpu.VMEM((C,), jnp.int32),
            pltpu.VMEM((L,), jnp.float32),
        ],
        compiler_params=pltpu.CompilerParams(needs_layout_passes=False),
    )
    return k(input)


# speedup vs baseline: 1.2680x; 1.2680x over previous
import functools

import jax
import jax.numpy as jnp
from jax import lax
from jax.experimental import pallas as pl
from jax.experimental.pallas import tpu as pltpu
from jax.experimental.pallas import tpu_sc as plsc

R, C = 128, 32768
H = C // 2
L = 16
NC, NS = 2, 16
NW = NC * NS
ROWS_PER_W = R // NW
U = 8


def _body(in_hbm, out_hbm, in0, in1, out0, out1, red16,
          isem0, isem1, osem0, osem1):
    wid = lax.axis_index("s") * NC + lax.axis_index("c")
    row0 = wid * ROWS_PER_W

    in_bufs = (in0, in1)
    in_sems = (isem0, isem1)
    out_bufs = (out0, out1)
    out_sems = (osem0, osem1)

    neg_inf = jnp.full((L,), -jnp.inf, jnp.float32)
    one = jnp.full((L,), 1, jnp.int32)
    zero = jnp.full((L,), 0, jnp.int32)

    n_chunks = C // (L * U)
    n_chunks_h = H // (L * U)

    in_dma = [None, None]
    out_dma = [None, None]

    in_dma[0] = pltpu.async_copy(in_hbm.at[row0], in0, isem0)

    for i in range(ROWS_PER_W):
        buf = in_bufs[i % 2]
        in_dma[i % 2].wait()
        if i + 1 < ROWS_PER_W:
            nxt = (i + 1) % 2
            in_dma[nxt] = pltpu.async_copy(
                in_hbm.at[row0 + i + 1], in_bufs[nxt], in_sems[nxt])

        def max_body(k, accs):
            base = k * (L * U)
            return tuple(
                jnp.maximum(a, buf[pl.ds(base + j * L, L)])
                for j, a in enumerate(accs)
            )

        accs = lax.fori_loop(0, n_chunks, max_body, (neg_inf,) * U)
        acc = functools.reduce(jnp.maximum, accs)
        srt, _ = plsc.sort_key_val(acc, acc)
        red16[...] = srt
        mxv = plsc.load_gather(red16, [jnp.full((L,), L - 1, jnp.int32)])

        for h in range(2):
            ob = out_bufs[h]
            if out_dma[h] is not None:
                out_dma[h].wait()

            def cmp_body(k, carry):
                src = h * H + k * (L * U)
                dst = k * (L * U)
                for j in range(U):
                    v = buf[pl.ds(src + j * L, L)]
                    ob[pl.ds(dst + j * L, L)] = jnp.where(v == mxv, one, zero)
                return carry

            lax.fori_loop(0, n_chunks_h, cmp_body, 0)
            out_dma[h] = pltpu.async_copy(
                ob, out_hbm.at[row0 + i, pl.ds(h * H, H)], out_sems[h])

    out_dma[0].wait()
    out_dma[1].wait()


def kernel(input):
    mesh = plsc.VectorSubcoreMesh(core_axis_name="c", subcore_axis_name="s")
    k = pl.kernel(
        _body,
        out_type=jax.ShapeDtypeStruct((R, C), jnp.int32),
        mesh=mesh,
        scratch_types=[
            pltpu.VMEM((C,), jnp.float32),
            pltpu.VMEM((C,), jnp.float32),
            pltpu.VMEM((H,), jnp.int32),
            pltpu.VMEM((H,), jnp.int32),
            pltpu.VMEM((L,), jnp.float32),
            pltpu.SemaphoreType.DMA,
            pltpu.SemaphoreType.DMA,
            pltpu.SemaphoreType.DMA,
            pltpu.SemaphoreType.DMA,
        ],
        compiler_params=pltpu.CompilerParams(needs_layout_passes=False),
    )
    return k(input)
